# initial kernel scaffold (unmeasured)
import jax
import jax.numpy as jnp
from jax import lax
from jax.experimental import pallas as pl
from jax.experimental.pallas import tpu as pltpu

N_DEV = 4
M, N = 8192, 4096
CHUNK = 512
SEG = CHUNK * N_DEV
HALF = M // 2
N_SEG = HALF // SEG
N_STEP = 2 * (N_DEV - 1)
TOTAL = N_SEG * N_STEP

_C = 0.7978845608028654


def _gelu(y):
    return 0.5 * y * (1.0 + jnp.tanh(_C * (y + 0.044715 * y * y * y)))


def kernel(x, w_mat):
    partial = jnp.dot(x, w_mat, preferred_element_type=jnp.float32)
    return _all_reduce_gelu(partial)


def _all_reduce_gelu(partial):
    def body(p_ref, out_ref,
             send0, recv0, stage0, send1, recv1, stage1,
             send_sems, recv_sems, stage_sems, out_sems, credit_sems):
        d = lax.axis_index("i")
        dirs = [
            dict(i=0, dirn=1, base=0, send=send0, recv=recv0, stage=stage0),
            dict(i=1, dirn=-1, base=HALF, send=send1, recv=recv1, stage=stage1),
        ]

        barrier = pltpu.get_barrier_semaphore()
        for nbr in [(d + 1) % N_DEV, (d - 1) % N_DEV]:
            pl.semaphore_signal(barrier, inc=1, device_id=(nbr,),
                                device_id_type=pl.DeviceIdType.MESH)
        pl.semaphore_wait(barrier, 2)

        def row_of(b, r, c):
            return b["base"] + r * SEG + c * CHUNK

        def stage_copy(b, r, c):
            return pltpu.make_async_copy(
                p_ref.at[pl.ds(row_of(b, r, c), CHUNK), :],
                b["stage"], stage_sems.at[b["i"]])

        for r in range(N_SEG):
            for b in dirs:
                cp = pltpu.make_async_copy(
                    p_ref.at[pl.ds(row_of(b, r, d), CHUNK), :],
                    b["send"], stage_sems.at[b["i"]])
                cp.start()
                cp.wait()
            for s in range(N_STEP):
                g = r * N_STEP + s
                rdmas = []
                for b in dirs:
                    nbr = (d + b["dirn"]) % N_DEV
                    if g > 0:
                        pl.semaphore_wait(credit_sems.at[b["i"]], 1)
                    rdma = pltpu.make_async_remote_copy(
                        src_ref=b["send"], dst_ref=b["recv"],
                        send_sem=send_sems.at[b["i"]],
                        recv_sem=recv_sems.at[b["i"]],
                        device_id=(nbr,),
                        device_id_type=pl.DeviceIdType.MESH)
                    rdma.start()
                    rdmas.append(rdma)
                    if s < N_DEV - 1:
                        stage_copy(b, r, (d - b["dirn"] * (s + 1)) % N_DEV).start()
                for b in dirs:
                    dirn = b["dirn"]
                    rdmas[b["i"]].wait()
                    if s < N_DEV - 1:
                        c = (d - dirn * (s + 1)) % N_DEV
                        stage_copy(b, r, c).wait()
                        y = b["recv"][:, :] + b["stage"][:, :]
                        if s == N_DEV - 2:
                            y = _gelu(y)
                            b["send"][:, :] = y
                            outcp = pltpu.make_async_copy(
                                b["send"],
                                out_ref.at[pl.ds(row_of(b, r, c), CHUNK), :],
                                out_sems.at[b["i"]])
                            outcp.start()
                            outcp.wait()
                        else:
                            b["send"][:, :] = y
                    else:
                        sa = s - (N_DEV - 1)
                        c = (d - dirn * sa) % N_DEV
                        outcp = pltpu.make_async_copy(
                            b["recv"],
                            out_ref.at[pl.ds(row_of(b, r, c), CHUNK), :],
                            out_sems.at[b["i"]])
                        outcp.start()
                        if s < N_STEP - 1:
                            b["send"][:, :] = b["recv"][:, :]
                        outcp.wait()
                    if g < TOTAL - 1:
                        pl.semaphore_signal(
                            credit_sems.at[b["i"]], inc=1,
                            device_id=((d - dirn) % N_DEV,),
                            device_id_type=pl.DeviceIdType.MESH)

    vmem = lambda: pltpu.VMEM((CHUNK, N), jnp.float32)
    return pl.pallas_call(
        body,
        out_shape=jax.ShapeDtypeStruct((M, N), jnp.float32),
        in_specs=[pl.BlockSpec(memory_space=pltpu.MemorySpace.ANY)],
        out_specs=pl.BlockSpec(memory_space=pltpu.MemorySpace.ANY),
        scratch_shapes=[
            vmem(), vmem(), vmem(),
            vmem(), vmem(), vmem(),
            pltpu.SemaphoreType.DMA((2,)),
            pltpu.SemaphoreType.DMA((2,)),
            pltpu.SemaphoreType.DMA((2,)),
            pltpu.SemaphoreType.DMA((2,)),
            pltpu.SemaphoreType.REGULAR((2,)),
        ],
        compiler_params=pltpu.CompilerParams(collective_id=0),
    )(partial)


# baseline (device time: 1463524 ns/iter reference)
import jax
import jax.numpy as jnp
from jax import lax
from jax.experimental import pallas as pl
from jax.experimental.pallas import tpu as pltpu

N_DEV = 4
M, N = 8192, 4096
CHUNK = 512
SEG = CHUNK * N_DEV
HALF = M // 2
N_SEG = HALF // SEG
N_STEP = 2 * (N_DEV - 1)
TOTAL = N_SEG * N_STEP

_C = 0.7978845608028654


def _gelu(y):
    return 0.5 * y * (1.0 + jnp.tanh(_C * (y + 0.044715 * y * y * y)))


def kernel(x, w_mat):
    partial = jnp.dot(x, w_mat, preferred_element_type=jnp.float32)
    return _all_reduce_gelu(partial)


def _all_reduce_gelu(partial):
    def body(p_ref, out_ref,
             send0, recv0, stage0, send1, recv1, stage1,
             send_sems, recv_sems, stage_sems, out_sems, credit_sems):
        d = lax.axis_index("i")
        dirs = [
            dict(i=0, dirn=1, base=0, send=send0, recv=recv0, stage=stage0),
            dict(i=1, dirn=-1, base=HALF, send=send1, recv=recv1, stage=stage1),
        ]

        barrier = pltpu.get_barrier_semaphore()
        for nbr in [(d + 1) % N_DEV, (d - 1) % N_DEV]:
            pl.semaphore_signal(barrier, inc=1, device_id=(nbr,),
                                device_id_type=pl.DeviceIdType.MESH)
        pl.semaphore_wait(barrier, 2)

        def row_of(b, r, c):
            return b["base"] + r * SEG + c * CHUNK

        def stage_copy(b, r, c):
            return pltpu.make_async_copy(
                p_ref.at[pl.ds(row_of(b, r, c), CHUNK), :],
                b["stage"], stage_sems.at[b["i"]])

        for r in range(N_SEG):
            for b in dirs:
                cp = pltpu.make_async_copy(
                    p_ref.at[pl.ds(row_of(b, r, d), CHUNK), :],
                    b["send"], stage_sems.at[b["i"]])
                cp.start()
                cp.wait()
            for s in range(N_STEP):
                g = r * N_STEP + s
                rdmas = []
                for b in dirs:
                    nbr = (d + b["dirn"]) % N_DEV
                    if g > 0:
                        pl.semaphore_wait(credit_sems.at[b["i"]], 1)
                    rdma = pltpu.make_async_remote_copy(
                        src_ref=b["send"], dst_ref=b["recv"],
                        send_sem=send_sems.at[b["i"]],
                        recv_sem=recv_sems.at[b["i"]],
                        device_id=(nbr,),
                        device_id_type=pl.DeviceIdType.MESH)
                    rdma.start()
                    rdmas.append(rdma)
                    if s < N_DEV - 1:
                        stage_copy(b, r, (d - b["dirn"] * (s + 1)) % N_DEV).start()
                for b in dirs:
                    dirn = b["dirn"]
                    rdmas[b["i"]].wait()
                    if s < N_DEV - 1:
                        c = (d - dirn * (s + 1)) % N_DEV
                        stage_copy(b, r, c).wait()
                        y = b["recv"][:, :] + b["stage"][:, :]
                        if s == N_DEV - 2:
                            y = _gelu(y)
                            b["send"][:, :] = y
                            outcp = pltpu.make_async_copy(
                                b["send"],
                                out_ref.at[pl.ds(row_of(b, r, c), CHUNK), :],
                                out_sems.at[b["i"]])
                            outcp.start()
                            outcp.wait()
                        else:
                            b["send"][:, :] = y
                    else:
                        sa = s - (N_DEV - 1)
                        c = (d - dirn * sa) % N_DEV
                        outcp = pltpu.make_async_copy(
                            b["recv"],
                            out_ref.at[pl.ds(row_of(b, r, c), CHUNK), :],
                            out_sems.at[b["i"]])
                        outcp.start()
                        if s < N_STEP - 1:
                            b["send"][:, :] = b["recv"][:, :]
                        outcp.wait()
                    if g < TOTAL - 1:
                        pl.semaphore_signal(
                            credit_sems.at[b["i"]], inc=1,
                            device_id=((d - dirn) % N_DEV,),
                            device_id_type=pl.DeviceIdType.MESH)

    vmem = lambda: pltpu.VMEM((CHUNK, N), jnp.float32)
    return pl.pallas_call(
        body,
        out_shape=jax.ShapeDtypeStruct((M, N), jnp.float32),
        in_specs=[pl.BlockSpec(memory_space=pl.ANY)],
        out_specs=pl.BlockSpec(memory_space=pl.ANY),
        scratch_shapes=[
            vmem(), vmem(), vmem(),
            vmem(), vmem(), vmem(),
            pltpu.SemaphoreType.DMA((2,)),
            pltpu.SemaphoreType.DMA((2,)),
            pltpu.SemaphoreType.DMA((2,)),
            pltpu.SemaphoreType.DMA((2,)),
            pltpu.SemaphoreType.REGULAR((2,)),
        ],
        compiler_params=pltpu.CompilerParams(
            collective_id=0, vmem_limit_bytes=60 * 1024 * 1024),
    )(partial)


# device time: 1326441 ns/iter; 1.1033x vs baseline; 1.1033x over previous
import jax
import jax.numpy as jnp
from jax import lax
from jax.experimental import pallas as pl
from jax.experimental.pallas import tpu as pltpu

N_DEV = 4
M, N = 8192, 4096
K_SHARD = 2048
CHUNK = 256
SEG = CHUNK * N_DEV
HALF = M // 2
N_SEG = HALF // SEG
N_STEP = 2 * (N_DEV - 1)
TOTAL = N_SEG * N_STEP
N_USE = N_SEG * N_DEV

_C = 0.7978845608028654


def _gelu(y):
    return 0.5 * y * (1.0 + jnp.tanh(_C * (y + 0.044715 * y * y * y)))


def kernel(x, w_mat):
    return _fused(x, w_mat.astype(jnp.bfloat16))


def _fused(x, w_bf):
    def body(x_ref, w_ref, out_ref,
             send0, recv0, part0, xs0,
             send1, recv1, part1, xs1,
             w_vmem,
             send_sems, recv_sems, x_sems, out_sems, credit_sems, w_sem):
        d = lax.axis_index("i")

        dirs = [
            dict(i=0, dirn=1, base=0, send=send0, recv=recv0,
                 part=part0, xs=xs0),
            dict(i=1, dirn=-1, base=HALF, send=send1, recv=recv1,
                 part=part1, xs=xs1),
        ]

        def use_rc(b, u):
            k = u - 1
            r = k // N_DEV
            us = k % N_DEV
            is_seed = us == N_DEV - 1
            r_dot = jnp.where(is_seed, r + 1, r)
            c = jnp.where(is_seed, d % N_DEV,
                          (d - b["dirn"] * (us + 1)) % N_DEV)
            return r_dot, c

        def xdma(b, u):
            r, c = use_rc(b, u)
            row0 = b["base"] + r * SEG + c * CHUNK
            return pltpu.make_async_copy(
                x_ref.at[pl.ds(row0, CHUNK), :],
                b["xs"].at[u % 2], x_sems.at[b["i"], u % 2])

        def do_dot(b, u):
            xdma(b, u).wait()
            res = jnp.dot(b["xs"][u % 2].astype(jnp.bfloat16), w_vmem[:, :],
                          preferred_element_type=jnp.float32)
            return res

        def rdma_desc(b, p):
            return pltpu.make_async_remote_copy(
                src_ref=b["send"], dst_ref=b["recv"].at[p],
                send_sem=send_sems.at[b["i"]],
                recv_sem=recv_sems.at[b["i"], p],
                device_id=((d + b["dirn"]) % N_DEV,),
                device_id_type=pl.DeviceIdType.MESH)

        def out_copy(b, row0, src):
            return pltpu.make_async_copy(
                src, out_ref.at[pl.ds(row0, CHUNK), :], out_sems.at[b["i"]])

        pltpu.make_async_copy(w_ref, w_vmem, w_sem).start()
        for b in dirs:
            xdma(b, 0).start()
            xdma(b, 1).start()
            pl.semaphore_signal(credit_sems.at[b["i"]], inc=2)
        barrier = pltpu.get_barrier_semaphore()
        for nbr in [(d + 1) % N_DEV, (d - 1) % N_DEV]:
            pl.semaphore_signal(barrier, inc=1, device_id=(nbr,),
                                device_id_type=pl.DeviceIdType.MESH)
        pl.semaphore_wait(barrier, 2)
        pltpu.make_async_copy(w_ref, w_vmem, w_sem).wait()

        for b in dirs:
            b["send"][...] = do_dot(b, 0)
            xdma(b, 2).start()

        def step(g, carry):
            r = g // N_STEP
            s = g - r * N_STEP
            p = g % 2
            u = 1 + N_DEV * r + jnp.where(s == N_STEP - 1,
                                          N_DEV - 1, s)

            for b in dirs:
                pl.semaphore_wait(credit_sems.at[b["i"]], 1)
                rdma_desc(b, p).start()

            dot_pred = jnp.logical_or(
                s < N_DEV - 1,
                jnp.logical_and(s == N_STEP - 1, r < N_SEG - 1))
            for b in dirs:
                @pl.when(dot_pred)
                def _(b=b, u=u):
                    b["part"][...] = do_dot(b, u)
                    @pl.when(u + 2 < N_USE)
                    def _():
                        xdma(b, u + 2).start()

            for b in dirs:
                dirn = b["dirn"]
                desc = rdma_desc(b, p)
                desc.wait_send()
                desc.wait_recv()

                @pl.when(s < N_DEV - 2)
                def _(b=b):
                    b["send"][...] = b["recv"][p] + b["part"][:, :]

                @pl.when(s == N_DEV - 2)
                def _(b=b):
                    c = (d - dirn * (s + 1)) % N_DEV
                    row0 = b["base"] + r * SEG + c * CHUNK
                    b["send"][...] = _gelu(b["recv"][p] + b["part"][:, :])
                    o = out_copy(b, row0, b["send"])
                    o.start()
                    o.wait()

                @pl.when(jnp.logical_and(s > N_DEV - 2, s < N_STEP - 1))
                def _(b=b):
                    c = (d - dirn * (s - (N_DEV - 1))) % N_DEV
                    row0 = b["base"] + r * SEG + c * CHUNK
                    b["send"][...] = b["recv"][p]
                    o = out_copy(b, row0, b["send"])
                    o.start()
                    o.wait()

                @pl.when(s == N_STEP - 1)
                def _(b=b):
                    c = (d - dirn * (s - (N_DEV - 1))) % N_DEV
                    row0 = b["base"] + r * SEG + c * CHUNK
                    o = out_copy(b, row0, b["recv"].at[p])
                    o.start()
                    b["send"][...] = b["part"][:, :]
                    o.wait()

                pl.semaphore_signal(
                    credit_sems.at[b["i"]], inc=1,
                    device_id=((d - dirn) % N_DEV,),
                    device_id_type=pl.DeviceIdType.MESH)
            return carry

        lax.fori_loop(0, TOTAL, step, 0)

        for b in dirs:
            pl.semaphore_wait(credit_sems.at[b["i"]], 2)

    return pl.pallas_call(
        body,
        out_shape=jax.ShapeDtypeStruct((M, N), jnp.float32),
        in_specs=[pl.BlockSpec(memory_space=pl.ANY),
                  pl.BlockSpec(memory_space=pl.ANY)],
        out_specs=pl.BlockSpec(memory_space=pl.ANY),
        scratch_shapes=[
            pltpu.VMEM((CHUNK, N), jnp.float32),
            pltpu.VMEM((2, CHUNK, N), jnp.float32),
            pltpu.VMEM((CHUNK, N), jnp.float32),
            pltpu.VMEM((2, CHUNK, K_SHARD), jnp.float32),
            pltpu.VMEM((CHUNK, N), jnp.float32),
            pltpu.VMEM((2, CHUNK, N), jnp.float32),
            pltpu.VMEM((CHUNK, N), jnp.float32),
            pltpu.VMEM((2, CHUNK, K_SHARD), jnp.float32),
            pltpu.VMEM((K_SHARD, N), jnp.bfloat16),
            pltpu.SemaphoreType.DMA((2,)),
            pltpu.SemaphoreType.DMA((2, 2)),
            pltpu.SemaphoreType.DMA((2, 2)),
            pltpu.SemaphoreType.DMA((2,)),
            pltpu.SemaphoreType.REGULAR((2,)),
            pltpu.SemaphoreType.DMA,
        ],
        compiler_params=pltpu.CompilerParams(
            collective_id=0, vmem_limit_bytes=64 * 1024 * 1024),
    )(x, w_bf)


# device time: 789996 ns/iter; 1.8526x vs baseline; 1.6790x over previous
import jax
import jax.numpy as jnp
from jax import lax
from jax.experimental import pallas as pl
from jax.experimental.pallas import tpu as pltpu

N_DEV = 4
M, N = 8192, 4096
K_SHARD = 2048
CHUNK = 256
SEG = CHUNK * N_DEV
HALF = M // 2
N_SEG = HALF // SEG
N_STEP = 2 * (N_DEV - 1)
TOTAL = N_SEG * N_STEP
N_USE = N_SEG * N_DEV

_C = 0.7978845608028654


def _gelu(y):
    return 0.5 * y * (1.0 + jnp.tanh(_C * (y + 0.044715 * y * y * y)))


def kernel(x, w_mat):
    return _fused(x, w_mat.astype(jnp.bfloat16))


def _fused(x, w_bf):
    def body(x_ref, w_ref, out_ref,
             send0, recv0, part0, xs0, ost0,
             send1, recv1, part1, xs1, ost1,
             w_vmem,
             send_sems, recv_sems, x_sems, out_sems, credit_sems, w_sem):
        d = lax.axis_index("i")

        dirs = [
            dict(i=0, dirn=1, base=0, send=send0, recv=recv0,
                 part=part0, xs=xs0, ost=ost0),
            dict(i=1, dirn=-1, base=HALF, send=send1, recv=recv1,
                 part=part1, xs=xs1, ost=ost1),
        ]

        def use_rc(b, u):
            k = u - 1
            r = k // N_DEV
            us = k % N_DEV
            is_seed = us == N_DEV - 1
            r_dot = jnp.where(is_seed, r + 1, r)
            c = jnp.where(is_seed, d % N_DEV,
                          (d - b["dirn"] * (us + 1)) % N_DEV)
            return r_dot, c

        def xdma(b, u):
            r, c = use_rc(b, u)
            row0 = b["base"] + r * SEG + c * CHUNK
            return pltpu.make_async_copy(
                x_ref.at[pl.ds(row0, CHUNK), :],
                b["xs"].at[u % 2], x_sems.at[b["i"], u % 2])

        def do_dot(b, u):
            xdma(b, u).wait()
            res = jnp.dot(b["xs"][u % 2].astype(jnp.bfloat16), w_vmem[:, :],
                          preferred_element_type=jnp.float32)
            return res

        def rdma_desc(b, p):
            return pltpu.make_async_remote_copy(
                src_ref=b["send"], dst_ref=b["recv"].at[p],
                send_sem=send_sems.at[b["i"]],
                recv_sem=recv_sems.at[b["i"], p],
                device_id=((d + b["dirn"]) % N_DEV,),
                device_id_type=pl.DeviceIdType.MESH)

        def out_copy(b, row0, src):
            return pltpu.make_async_copy(
                src, out_ref.at[pl.ds(row0, CHUNK), :], out_sems.at[b["i"]])

        pltpu.make_async_copy(w_ref, w_vmem, w_sem).start()
        for b in dirs:
            xdma(b, 0).start()
            xdma(b, 1).start()
            pl.semaphore_signal(credit_sems.at[b["i"]], inc=2)
        barrier = pltpu.get_barrier_semaphore()
        for nbr in [(d + 1) % N_DEV, (d - 1) % N_DEV]:
            pl.semaphore_signal(barrier, inc=1, device_id=(nbr,),
                                device_id_type=pl.DeviceIdType.MESH)
        pl.semaphore_wait(barrier, 2)
        pltpu.make_async_copy(w_ref, w_vmem, w_sem).wait()

        for b in dirs:
            b["send"][...] = do_dot(b, 0).astype(jnp.bfloat16)
            xdma(b, 2).start()

        def step(g, carry):
            r = g // N_STEP
            s = g - r * N_STEP
            p = g % 2
            u = 1 + N_DEV * r + jnp.where(s == N_STEP - 1,
                                          N_DEV - 1, s)

            for b in dirs:
                pl.semaphore_wait(credit_sems.at[b["i"]], 1)
                rdma_desc(b, p).start()

            dot_pred = jnp.logical_or(
                s < N_DEV - 1,
                jnp.logical_and(s == N_STEP - 1, r < N_SEG - 1))
            for b in dirs:
                @pl.when(dot_pred)
                def _(b=b, u=u):
                    b["part"][...] = do_dot(b, u)
                    @pl.when(u + 2 < N_USE)
                    def _():
                        xdma(b, u + 2).start()

            for b in dirs:
                dirn = b["dirn"]
                desc = rdma_desc(b, p)
                desc.wait_send()
                desc.wait_recv()

                @pl.when(s < N_DEV - 2)
                def _(b=b):
                    acc = b["recv"][p].astype(jnp.float32) + b["part"][:, :]
                    b["send"][...] = acc.astype(jnp.bfloat16)

                @pl.when(s == N_DEV - 2)
                def _(b=b):
                    c = (d - dirn * (s + 1)) % N_DEV
                    row0 = b["base"] + r * SEG + c * CHUNK
                    y = _gelu(b["recv"][p].astype(jnp.float32)
                              + b["part"][:, :])
                    b["send"][...] = y.astype(jnp.bfloat16)
                    b["ost"][...] = y
                    o = out_copy(b, row0, b["ost"])
                    o.start()
                    o.wait()

                @pl.when(jnp.logical_and(s > N_DEV - 2, s < N_STEP - 1))
                def _(b=b):
                    c = (d - dirn * (s - (N_DEV - 1))) % N_DEV
                    row0 = b["base"] + r * SEG + c * CHUNK
                    b["send"][...] = b["recv"][p]
                    b["ost"][...] = b["recv"][p].astype(jnp.float32)
                    o = out_copy(b, row0, b["ost"])
                    o.start()
                    o.wait()

                @pl.when(s == N_STEP - 1)
                def _(b=b):
                    c = (d - dirn * (s - (N_DEV - 1))) % N_DEV
                    row0 = b["base"] + r * SEG + c * CHUNK
                    b["ost"][...] = b["recv"][p].astype(jnp.float32)
                    o = out_copy(b, row0, b["ost"])
                    o.start()
                    b["send"][...] = b["part"][:, :].astype(jnp.bfloat16)
                    o.wait()

                pl.semaphore_signal(
                    credit_sems.at[b["i"]], inc=1,
                    device_id=((d - dirn) % N_DEV,),
                    device_id_type=pl.DeviceIdType.MESH)
            return carry

        lax.fori_loop(0, TOTAL, step, 0)

        for b in dirs:
            pl.semaphore_wait(credit_sems.at[b["i"]], 2)

    return pl.pallas_call(
        body,
        out_shape=jax.ShapeDtypeStruct((M, N), jnp.float32),
        in_specs=[pl.BlockSpec(memory_space=pl.ANY),
                  pl.BlockSpec(memory_space=pl.ANY)],
        out_specs=pl.BlockSpec(memory_space=pl.ANY),
        scratch_shapes=[
            pltpu.VMEM((CHUNK, N), jnp.bfloat16),
            pltpu.VMEM((2, CHUNK, N), jnp.bfloat16),
            pltpu.VMEM((CHUNK, N), jnp.float32),
            pltpu.VMEM((2, CHUNK, K_SHARD), jnp.float32),
            pltpu.VMEM((CHUNK, N), jnp.float32),
            pltpu.VMEM((CHUNK, N), jnp.bfloat16),
            pltpu.VMEM((2, CHUNK, N), jnp.bfloat16),
            pltpu.VMEM((CHUNK, N), jnp.float32),
            pltpu.VMEM((2, CHUNK, K_SHARD), jnp.float32),
            pltpu.VMEM((CHUNK, N), jnp.float32),
            pltpu.VMEM((K_SHARD, N), jnp.bfloat16),
            pltpu.SemaphoreType.DMA((2,)),
            pltpu.SemaphoreType.DMA((2, 2)),
            pltpu.SemaphoreType.DMA((2, 2)),
            pltpu.SemaphoreType.DMA((2,)),
            pltpu.SemaphoreType.REGULAR((2,)),
            pltpu.SemaphoreType.DMA,
        ],
        compiler_params=pltpu.CompilerParams(
            collective_id=0, vmem_limit_bytes=64 * 1024 * 1024),
    )(x, w_bf)


# device time: 735462 ns/iter; 1.9899x vs baseline; 1.0741x over previous
import jax
import jax.numpy as jnp
from jax import lax
from jax.experimental import pallas as pl
from jax.experimental.pallas import tpu as pltpu

N_DEV = 4
M, N = 8192, 4096
K_SHARD = 2048
CHUNK = 256
SEG = CHUNK * N_DEV
HALF = M // 2
N_SEG = HALF // SEG
N_STEP = 2 * (N_DEV - 1)
TOTAL = N_SEG * N_STEP
N_USE = N_SEG * N_DEV

_C = 0.7978845608028654


def _gelu(y):
    return 0.5 * y * (1.0 + jnp.tanh(_C * (y + 0.044715 * y * y * y)))


def kernel(x, w_mat):
    return _fused(x, w_mat.astype(jnp.bfloat16))


def _fused(x, w_bf):
    def body(x_ref, w_ref, out_ref,
             send0, recv0, part0, xs0, ost0,
             send1, recv1, part1, xs1, ost1,
             w_vmem,
             send_sems, recv_sems, x_sems, out_sems, credit_sems, w_sem):
        d = lax.axis_index("i")

        dirs = [
            dict(i=0, dirn=1, base=0, send=send0, recv=recv0,
                 part=part0, xs=xs0, ost=ost0),
            dict(i=1, dirn=-1, base=HALF, send=send1, recv=recv1,
                 part=part1, xs=xs1, ost=ost1),
        ]

        def use_rc(b, u):
            k = u - 1
            r = k // N_DEV
            us = k % N_DEV
            is_seed = us == N_DEV - 1
            r_dot = jnp.where(is_seed, r + 1, r)
            c = jnp.where(is_seed, d % N_DEV,
                          (d - b["dirn"] * (us + 1)) % N_DEV)
            return r_dot, c

        def xdma(b, u):
            r, c = use_rc(b, u)
            row0 = b["base"] + r * SEG + c * CHUNK
            return pltpu.make_async_copy(
                x_ref.at[pl.ds(row0, CHUNK), :],
                b["xs"].at[u % 2], x_sems.at[b["i"], u % 2])

        def do_dot(b, u):
            xdma(b, u).wait()
            res = jnp.dot(b["xs"][u % 2].astype(jnp.bfloat16), w_vmem[:, :],
                          preferred_element_type=jnp.float32)
            return res

        def rdma_desc(b, p):
            return pltpu.make_async_remote_copy(
                src_ref=b["send"], dst_ref=b["recv"].at[p],
                send_sem=send_sems.at[b["i"]],
                recv_sem=recv_sems.at[b["i"], p],
                device_id=((d + b["dirn"]) % N_DEV,),
                device_id_type=pl.DeviceIdType.MESH)

        def site_chunk(b, site):
            return (d - b["dirn"] * (site - 1)) % N_DEV

        def ost_copy(b, r, site):
            row0 = b["base"] + r * SEG + site_chunk(b, site) * CHUNK
            return pltpu.make_async_copy(
                b["ost"], out_ref.at[pl.ds(row0, CHUNK), :],
                out_sems.at[b["i"], site])

        pltpu.make_async_copy(w_ref, w_vmem, w_sem).start()
        for b in dirs:
            xdma(b, 0).start()
            xdma(b, 1).start()
            pl.semaphore_signal(credit_sems.at[b["i"]], inc=2)
        barrier = pltpu.get_barrier_semaphore()
        for nbr in [(d + 1) % N_DEV, (d - 1) % N_DEV]:
            pl.semaphore_signal(barrier, inc=1, device_id=(nbr,),
                                device_id_type=pl.DeviceIdType.MESH)
        pl.semaphore_wait(barrier, 2)
        pltpu.make_async_copy(w_ref, w_vmem, w_sem).wait()

        for b in dirs:
            b["send"][...] = do_dot(b, 0).astype(jnp.bfloat16)
            xdma(b, 2).start()

        def step(g, carry):
            r = g // N_STEP
            s = g - r * N_STEP
            p = g % 2
            u = 1 + N_DEV * r + jnp.where(s == N_STEP - 1,
                                          N_DEV - 1, s)

            for b in dirs:
                pl.semaphore_wait(credit_sems.at[b["i"]], 1)
                rdma_desc(b, p).start()

            dot_pred = jnp.logical_or(
                s < N_DEV - 1,
                jnp.logical_and(s == N_STEP - 1, r < N_SEG - 1))
            for b in dirs:
                @pl.when(dot_pred)
                def _(b=b, u=u):
                    b["part"][...] = do_dot(b, u)
                    @pl.when(u + 2 < N_USE)
                    def _():
                        xdma(b, u + 2).start()

            for b in dirs:
                dirn = b["dirn"]
                desc = rdma_desc(b, p)
                desc.wait_send()
                desc.wait_recv()

                @pl.when(s < N_DEV - 2)
                def _(b=b):
                    acc = b["recv"][p].astype(jnp.float32) + b["part"][:, :]
                    b["send"][...] = acc.astype(jnp.bfloat16)

                @pl.when(s == N_DEV - 2)
                def _(b=b):
                    @pl.when(r > 0)
                    def _():
                        ost_copy(b, r - 1, 3).wait()
                    y = _gelu(b["recv"][p].astype(jnp.float32)
                              + b["part"][:, :])
                    b["send"][...] = y.astype(jnp.bfloat16)
                    b["ost"][...] = y
                    ost_copy(b, r, 0).start()

                @pl.when(jnp.logical_and(s > N_DEV - 2, s < N_STEP - 1))
                def _(b=b):
                    site = s - 2
                    ost_copy(b, r, site - 1).wait()
                    b["send"][...] = b["recv"][p]
                    b["ost"][...] = b["recv"][p].astype(jnp.float32)
                    ost_copy(b, r, site).start()

                @pl.when(s == N_STEP - 1)
                def _(b=b):
                    ost_copy(b, r, 2).wait()
                    b["ost"][...] = b["recv"][p].astype(jnp.float32)
                    ost_copy(b, r, 3).start()
                    b["send"][...] = b["part"][:, :].astype(jnp.bfloat16)

                pl.semaphore_signal(
                    credit_sems.at[b["i"]], inc=1,
                    device_id=((d - dirn) % N_DEV,),
                    device_id_type=pl.DeviceIdType.MESH)
            return carry

        lax.fori_loop(0, TOTAL, step, 0)

        for b in dirs:
            ost_copy(b, N_SEG - 1, 3).wait()
            pl.semaphore_wait(credit_sems.at[b["i"]], 2)

    return pl.pallas_call(
        body,
        out_shape=jax.ShapeDtypeStruct((M, N), jnp.float32),
        in_specs=[pl.BlockSpec(memory_space=pl.ANY),
                  pl.BlockSpec(memory_space=pl.ANY)],
        out_specs=pl.BlockSpec(memory_space=pl.ANY),
        scratch_shapes=[
            pltpu.VMEM((CHUNK, N), jnp.bfloat16),
            pltpu.VMEM((2, CHUNK, N), jnp.bfloat16),
            pltpu.VMEM((CHUNK, N), jnp.float32),
            pltpu.VMEM((2, CHUNK, K_SHARD), jnp.float32),
            pltpu.VMEM((CHUNK, N), jnp.float32),
            pltpu.VMEM((CHUNK, N), jnp.bfloat16),
            pltpu.VMEM((2, CHUNK, N), jnp.bfloat16),
            pltpu.VMEM((CHUNK, N), jnp.float32),
            pltpu.VMEM((2, CHUNK, K_SHARD), jnp.float32),
            pltpu.VMEM((CHUNK, N), jnp.float32),
            pltpu.VMEM((K_SHARD, N), jnp.bfloat16),
            pltpu.SemaphoreType.DMA((2,)),
            pltpu.SemaphoreType.DMA((2, 2)),
            pltpu.SemaphoreType.DMA((2, 2)),
            pltpu.SemaphoreType.DMA((2, 4)),
            pltpu.SemaphoreType.REGULAR((2,)),
            pltpu.SemaphoreType.DMA,
        ],
        compiler_params=pltpu.CompilerParams(
            collective_id=0, vmem_limit_bytes=64 * 1024 * 1024),
    )(x, w_bf)
